# baseline (speedup 1.0000x reference)
import functools

import jax
import jax.numpy as jnp
from jax import lax
from jax.experimental import pallas as pl
from jax.experimental.pallas import tpu as pltpu
from jax.experimental.pallas import tpu_sc as plsc

B = 16384
V = 100000
D = 32
NUM_DIM = 16
EMB_DIM = 128
NUM_OUT = 64

NC = 2
NS = 16
NW = NC * NS

CH = 4096
NCHK = B // CH


def _sc_gather_body(tt0, tt1, tt2, tt3, c0, c1, c2, c3,
                    out, idx_v, col_v, out_v, isem, osem):
    wid = lax.axis_index("s") * NC + lax.axis_index("c")
    tabs = (tt0, tt1, tt2, tt3)
    cats = (c0, c1, c2, c3)
    out_cp = None
    for t in range(4):
        col_cp = pltpu.async_copy(tabs[t].at[wid], col_v, isem)
        chunk_cps = [pltpu.async_copy(
            cats[t].at[pl.ds(0, CH)], idx_v.at[0], isem)]
        col_cp.wait()
        if out_cp is not None:
            out_cp.wait()
        for ch in range(NCHK):
            if ch + 1 < NCHK:
                chunk_cps.append(pltpu.async_copy(
                    cats[t].at[pl.ds((ch + 1) * CH, CH)],
                    idx_v.at[(ch + 1) % 2], isem))
            chunk_cps[ch].wait()

            @plsc.parallel_loop(0, CH // 16, unroll=16)
            def _(j, ch=ch, buf=ch % 2):
                off = j * 16
                iv = idx_v[buf, pl.ds(off, 16)]
                out_v[pl.ds(ch * CH + off, 16)] = (
                    plsc.load_gather(col_v, [iv]))
        out_cp = pltpu.async_copy(out_v, out.at[t * D + wid], osem)
    out_cp.wait()


@jax.jit
def _sc_gather(tt0, tt1, tt2, tt3, c0, c1, c2, c3):
    mesh = plsc.VectorSubcoreMesh(core_axis_name="c", subcore_axis_name="s")
    f = functools.partial(
        pl.kernel,
        mesh=mesh,
        out_type=jax.ShapeDtypeStruct((4 * D, B), jnp.float32),
        scratch_types=[
            pltpu.VMEM((2, CH), jnp.int32),
            pltpu.VMEM((V,), jnp.float32),
            pltpu.VMEM((B,), jnp.float32),
            pltpu.SemaphoreType.DMA,
            pltpu.SemaphoreType.DMA,
        ],
        compiler_params=pltpu.CompilerParams(needs_layout_passes=False),
    )(_sc_gather_body)
    return f(tt0, tt1, tt2, tt3, c0, c1, c2, c3)


def _dot_nt(a, b):
    return lax.dot_general(a, b, (((1,), (1,)), ((), ())),
                           preferred_element_type=jnp.float32)


def _stats_body(xf_ref, nw_ref, nb_ref, g_ref, bt_ref, s_ref, t_ref):
    num = _dot_nt(xf_ref[...], nw_ref[...]) + nb_ref[...]
    mean = jnp.mean(num, axis=0, keepdims=True)
    var = jnp.mean((num - mean) ** 2, axis=0, keepdims=True)
    s = g_ref[...] * lax.rsqrt(var + 1e-5)
    s_ref[...] = s
    t_ref[...] = bt_ref[...] - mean * s


@jax.jit
def _tc_stats(x, num_W, nb, g, bt):
    return pl.pallas_call(
        _stats_body,
        out_shape=[jax.ShapeDtypeStruct((1, NUM_OUT), jnp.float32)] * 2,
    )(x, num_W, nb, g, bt)


def _tc_fuse_body(e_ref, x_ref, nw_ref, nb_ref, s_ref, t_ref,
                  fw_ref, fb_ref, out_ref):
    num_blk = _dot_nt(x_ref[...], nw_ref[...]) + nb_ref[...]
    nn = num_blk * s_ref[...] + t_ref[...]
    fw = fw_ref[...]
    acc = _dot_nt(nn, fw[:, 4 * D:])
    acc += lax.dot_general(e_ref[...], fw[:, :4 * D],
                           (((0,), (1,)), ((), ())),
                           preferred_element_type=jnp.float32)
    out_ref[...] = acc + fb_ref[...]


BT = 16384


@jax.jit
def _tc_fuse(e, x, num_W, nb, s, t, fw, fb):
    grid = (B // BT,)
    whole = lambda shape: pl.BlockSpec(shape, lambda i: (0, 0))
    blk = lambda shape: pl.BlockSpec(shape, lambda i: (i, 0))
    return pl.pallas_call(
        _tc_fuse_body,
        grid=grid,
        in_specs=[
            pl.BlockSpec((4 * D, BT), lambda i: (0, i)),
            blk((BT, NUM_DIM)),
            whole((NUM_OUT, NUM_DIM)),
            whole((1, NUM_OUT)),
            whole((1, NUM_OUT)),
            whole((1, NUM_OUT)),
            whole((EMB_DIM, 4 * D + NUM_OUT)),
            whole((1, EMB_DIM)),
        ],
        out_specs=blk((BT, EMB_DIM)),
        out_shape=jax.ShapeDtypeStruct((B, EMB_DIM), jnp.float32),
    )(e, x, num_W, nb, s, t, fw, fb)


def kernel(cat_0, cat_1, cat_2, cat_3, numerical_features,
           table_0, table_1, table_2, table_3,
           num_W, num_b, bn_gamma, bn_beta, final_W, final_b):
    cats = [c.astype(jnp.int32) for c in (cat_0, cat_1, cat_2, cat_3)]
    tts = [jnp.transpose(t) for t in (table_0, table_1, table_2, table_3)]
    e = _sc_gather(*tts, *cats)
    s, t = _tc_stats(numerical_features, num_W, num_b.reshape(1, -1),
                     bn_gamma.reshape(1, -1), bn_beta.reshape(1, -1))
    return _tc_fuse(e, numerical_features, num_W, num_b.reshape(1, -1),
                    s, t, final_W, final_b.reshape(1, -1))


# speedup vs baseline: 1.0462x; 1.0462x over previous
import functools

import jax
import jax.numpy as jnp
from jax import lax
from jax.experimental import pallas as pl
from jax.experimental.pallas import tpu as pltpu
from jax.experimental.pallas import tpu_sc as plsc

B = 16384
V = 100000
D = 32
NUM_DIM = 16
EMB_DIM = 128
NUM_OUT = 64

NC = 2
NS = 16
NW = NC * NS

CH = 4096
NCHK = B // CH


def _sc_gather_body(tt0, tt1, tt2, tt3, c0, c1, c2, c3,
                    out, idx_v, col_v, out_v, isem, osem):
    wid = lax.axis_index("s") * NC + lax.axis_index("c")
    tabs = (tt0, tt1, tt2, tt3)
    cats = (c0, c1, c2, c3)
    out_cp = None
    for t in range(4):
        col_cp = pltpu.async_copy(tabs[t].at[wid], col_v, isem)
        chunk_cps = [pltpu.async_copy(
            cats[t].at[pl.ds(0, CH)], idx_v.at[0], isem)]
        col_cp.wait()
        if out_cp is not None:
            out_cp.wait()
        for ch in range(NCHK):
            if ch + 1 < NCHK:
                chunk_cps.append(pltpu.async_copy(
                    cats[t].at[pl.ds((ch + 1) * CH, CH)],
                    idx_v.at[(ch + 1) % 2], isem))
            chunk_cps[ch].wait()

            @plsc.parallel_loop(0, CH // 16, unroll=16)
            def _(j, ch=ch, buf=ch % 2):
                off = j * 16
                iv = idx_v[buf, pl.ds(off, 16)]
                out_v[pl.ds(ch * CH + off, 16)] = (
                    plsc.load_gather(col_v, [iv]))
        out_cp = pltpu.async_copy(out_v, out.at[t * D + wid], osem)
    out_cp.wait()


@jax.jit
def _sc_gather(tt0, tt1, tt2, tt3, c0, c1, c2, c3):
    mesh = plsc.VectorSubcoreMesh(core_axis_name="c", subcore_axis_name="s")
    f = functools.partial(
        pl.kernel,
        mesh=mesh,
        out_type=jax.ShapeDtypeStruct((4 * D, B), jnp.float32),
        scratch_types=[
            pltpu.VMEM((2, CH), jnp.int32),
            pltpu.VMEM((V,), jnp.float32),
            pltpu.VMEM((B,), jnp.float32),
            pltpu.SemaphoreType.DMA,
            pltpu.SemaphoreType.DMA,
        ],
        compiler_params=pltpu.CompilerParams(needs_layout_passes=False),
    )(_sc_gather_body)
    return f(tt0, tt1, tt2, tt3, c0, c1, c2, c3)


def _dot_nt(a, b):
    return lax.dot_general(a, b, (((1,), (1,)), ((), ())),
                           preferred_element_type=jnp.float32)


def _stats_body(xf_ref, nw_ref, nb_ref, g_ref, bt_ref, s_ref, t_ref):
    num = _dot_nt(xf_ref[...], nw_ref[...]) + nb_ref[...]
    mean = jnp.mean(num, axis=0, keepdims=True)
    var = jnp.mean((num - mean) ** 2, axis=0, keepdims=True)
    s = g_ref[...] * lax.rsqrt(var + 1e-5)
    s_ref[...] = s
    t_ref[...] = bt_ref[...] - mean * s


@jax.jit
def _tc_stats(x, num_W, nb, g, bt):
    return pl.pallas_call(
        _stats_body,
        out_shape=[jax.ShapeDtypeStruct((1, NUM_OUT), jnp.float32)] * 2,
    )(x, num_W, nb, g, bt)


def _tc_fuse_body(e_ref, x_ref, nw_ref, nb_ref, s_ref, t_ref,
                  fw_ref, fb_ref, out_ref):
    num_blk = _dot_nt(x_ref[...], nw_ref[...]) + nb_ref[...]
    nn = num_blk * s_ref[...] + t_ref[...]
    fw = fw_ref[...]
    acc = _dot_nt(nn, fw[:, 4 * D:])
    acc += lax.dot_general(e_ref[...], fw[:, :4 * D],
                           (((0,), (1,)), ((), ())),
                           preferred_element_type=jnp.float32)
    out_ref[...] = acc + fb_ref[...]


BT = 8192


@jax.jit
def _tc_fuse(e, x, num_W, nb, s, t, fw, fb):
    grid = (B // BT,)
    whole = lambda shape: pl.BlockSpec(shape, lambda i: (0, 0))
    blk = lambda shape: pl.BlockSpec(shape, lambda i: (i, 0))
    return pl.pallas_call(
        _tc_fuse_body,
        grid=grid,
        in_specs=[
            pl.BlockSpec((4 * D, BT), lambda i: (0, i)),
            blk((BT, NUM_DIM)),
            whole((NUM_OUT, NUM_DIM)),
            whole((1, NUM_OUT)),
            whole((1, NUM_OUT)),
            whole((1, NUM_OUT)),
            whole((EMB_DIM, 4 * D + NUM_OUT)),
            whole((1, EMB_DIM)),
        ],
        out_specs=blk((BT, EMB_DIM)),
        out_shape=jax.ShapeDtypeStruct((B, EMB_DIM), jnp.float32),
    )(e, x, num_W, nb, s, t, fw, fb)


def kernel(cat_0, cat_1, cat_2, cat_3, numerical_features,
           table_0, table_1, table_2, table_3,
           num_W, num_b, bn_gamma, bn_beta, final_W, final_b):
    cats = [c.astype(jnp.int32) for c in (cat_0, cat_1, cat_2, cat_3)]
    tts = [jnp.transpose(t) for t in (table_0, table_1, table_2, table_3)]
    e = _sc_gather(*tts, *cats)
    s, t = _tc_stats(numerical_features, num_W, num_b.reshape(1, -1),
                     bn_gamma.reshape(1, -1), bn_beta.reshape(1, -1))
    return _tc_fuse(e, numerical_features, num_W, num_b.reshape(1, -1),
                    s, t, final_W, final_b.reshape(1, -1))
